# baseline (speedup 1.0000x reference)
import functools

import jax
import jax.numpy as jnp
from jax import lax
from jax.experimental import pallas as pl
from jax.experimental.pallas import tpu as pltpu
from jax.experimental.pallas import tpu_sc as plsc

H = 1024
NSP = 8
CORPUS = 512
ROWS = CORPUS + NSP
BATCH = 32
W2_ROWS = NSP * H
W2_BLK = 2048
N_WORKERS = 32
N_SLICES = 8
SLICE_ROWS = CORPUS // N_SLICES
TILES_PER_SLICE = N_WORKERS // N_SLICES
BATCH_PER_TILE = BATCH // TILES_PER_SLICE


def _mlp_body(w1_ref, b1_ref, w2_ref, b2_ref, spt_ref, h_ref):
    r = pl.program_id(0)

    @pl.when(r == 0)
    def _():
        ones = jnp.ones((H, 8), jnp.float32)
        hc = lax.dot_general(w1_ref[...], ones, (((1,), (0,)), ((), ())),
                             preferred_element_type=jnp.float32)
        h_ref[...] = jax.nn.relu(hc + b1_ref[...])

    res = lax.dot_general(w2_ref[...], h_ref[...], (((1,), (0,)), ((), ())),
                          preferred_element_type=jnp.float32)
    spt_ref[...] = res + b2_ref[...]


def _soft_prompt(W1, b1, W2, b2):
    b1r = b1.reshape(H, 1)
    b2r = b2.reshape(W2_ROWS, 1)
    spt = pl.pallas_call(
        _mlp_body,
        grid=(W2_ROWS // W2_BLK,),
        in_specs=[
            pl.BlockSpec((H, H), lambda r: (0, 0)),
            pl.BlockSpec((H, 1), lambda r: (0, 0)),
            pl.BlockSpec((W2_BLK, H), lambda r: (r, 0)),
            pl.BlockSpec((W2_BLK, 1), lambda r: (r, 0)),
        ],
        out_specs=pl.BlockSpec((W2_BLK, 8), lambda r: (r, 0)),
        out_shape=jax.ShapeDtypeStruct((W2_ROWS, 8), jnp.float32),
        scratch_shapes=[pltpu.VMEM((H, 8), jnp.float32)],
    )(W1, b1r, W2, b2r)
    return spt[:, 0].reshape(NSP, H)


def _sc_broadcast_de(de):
    mesh = plsc.VectorSubcoreMesh(core_axis_name="c", subcore_axis_name="s")

    @functools.partial(
        pl.kernel,
        out_type=jax.ShapeDtypeStruct((BATCH, ROWS, H), jnp.float32),
        mesh=mesh,
        scratch_types=[
            pltpu.VMEM((SLICE_ROWS, H), jnp.float32),
            pltpu.SemaphoreType.DMA,
        ],
    )
    def body(de_hbm, out_hbm, local, sem):
        c = lax.axis_index("c")
        s = lax.axis_index("s")
        wid = s * 2 + c
        slice_id = wid // TILES_PER_SLICE
        lane = wid % TILES_PER_SLICE
        r0 = slice_id * SLICE_ROWS
        pltpu.sync_copy(de_hbm.at[pl.ds(r0, SLICE_ROWS)], local)
        copies = []
        for i in range(BATCH_PER_TILE):
            b = lane * BATCH_PER_TILE + i
            copies.append(
                pltpu.async_copy(local, out_hbm.at[b, pl.ds(r0, SLICE_ROWS)], sem))
        for cp in copies:
            cp.wait()

    return body(de)


def _sp_write_body(out_alias_ref, sp_ref, out_ref):
    del out_alias_ref
    out_ref[...] = jnp.broadcast_to(sp_ref[...][None], (BATCH, NSP, H))


def _sp_write(out1, sp):
    return pl.pallas_call(
        _sp_write_body,
        grid=(1,),
        in_specs=[
            pl.BlockSpec(memory_space=pl.ANY),
            pl.BlockSpec((NSP, H), lambda i: (0, 0)),
        ],
        out_specs=pl.BlockSpec((BATCH, NSP, H), lambda i: (0, 64, 0)),
        out_shape=jax.ShapeDtypeStruct((BATCH, ROWS, H), jnp.float32),
        input_output_aliases={0: 0},
    )(out1, sp)


def kernel(input_ids, dataset_embeddings, W1, b1, W2, b2):
    del input_ids
    de = dataset_embeddings.astype(jnp.float32)
    sp = _soft_prompt(W1, b1, W2, b2)
    out = _sc_broadcast_de(de)
    return _sp_write(out, sp)


# speedup vs baseline: 1.0936x; 1.0936x over previous
import functools

import jax
import jax.numpy as jnp
from jax import lax
from jax.experimental import pallas as pl
from jax.experimental.pallas import tpu as pltpu
from jax.experimental.pallas import tpu_sc as plsc

H = 1024
NSP = 8
CORPUS = 512
ROWS = CORPUS + NSP
BATCH = 32
SC_BATCHES = 16
W2_ROWS = NSP * H


def _mlp_body(w1_ref, b1_ref, w2_ref, b2_ref, sp_ref, h_ref):
    r = pl.program_id(0)

    @pl.when(r == 0)
    def _():
        ones = jnp.ones((8, H), jnp.float32)
        h = lax.dot_general(ones, w1_ref[...], (((1,), (1,)), ((), ())),
                            preferred_element_type=jnp.float32)
        h_ref[...] = jax.nn.relu(h + b1_ref[...])

    res = lax.dot_general(h_ref[...], w2_ref[...], (((1,), (1,)), ((), ())),
                          preferred_element_type=jnp.float32)
    sp_ref[pl.ds(r, 1), :] = res[0:1, :] + b2_ref[pl.ds(r, 1), :]


def _soft_prompt(W1, b1, W2, b2):
    b1r = b1.reshape(1, H)
    b2r = b2.reshape(NSP, H)
    return pl.pallas_call(
        _mlp_body,
        grid=(NSP,),
        in_specs=[
            pl.BlockSpec((H, H), lambda r: (0, 0)),
            pl.BlockSpec((1, H), lambda r: (0, 0)),
            pl.BlockSpec((H, H), lambda r: (r, 0)),
            pl.BlockSpec((NSP, H), lambda r: (0, 0)),
        ],
        out_specs=pl.BlockSpec((NSP, H), lambda r: (0, 0)),
        out_shape=jax.ShapeDtypeStruct((NSP, H), jnp.float32),
        scratch_shapes=[pltpu.VMEM((8, H), jnp.float32)],
    )(W1, b1r, W2, b2r)


def _sc_broadcast_de(de):
    mesh = plsc.ScalarSubcoreMesh(axis_name="c", num_cores=2)
    per_core = SC_BATCHES // 2

    @functools.partial(
        pl.kernel,
        out_type=jax.ShapeDtypeStruct((BATCH, ROWS, H), jnp.float32),
        mesh=mesh,
        scratch_types=[
            pltpu.VMEM_SHARED((CORPUS, H), jnp.float32),
            pltpu.SemaphoreType.DMA,
        ],
    )
    def body(de_hbm, out_hbm, shared, sem):
        c = lax.axis_index("c")
        pltpu.sync_copy(de_hbm, shared)
        copies = []
        for i in range(per_core):
            b = c * per_core + i
            copies.append(
                pltpu.async_copy(shared, out_hbm.at[b, pl.ds(0, CORPUS)], sem))
        for cp in copies:
            cp.wait()

    return body(de)


def _sp_write_body(out_alias_ref, sp_ref, out_ref):
    del out_alias_ref
    out_ref[...] = jnp.broadcast_to(sp_ref[...][None], (BATCH, NSP, H))


def _sp_write(out1, sp):
    return pl.pallas_call(
        _sp_write_body,
        grid=(1,),
        in_specs=[
            pl.BlockSpec(memory_space=pl.ANY),
            pl.BlockSpec((NSP, H), lambda i: (0, 0)),
        ],
        out_specs=pl.BlockSpec((BATCH, NSP, H), lambda i: (0, 64, 0)),
        out_shape=jax.ShapeDtypeStruct((BATCH, ROWS, H), jnp.float32),
        input_output_aliases={0: 0},
    )(out1, sp)


def _de_write_body(out_alias_ref, de_ref, out_ref):
    del out_alias_ref
    out_ref[...] = de_ref[...][None]


def _tc_de_write(out1, de):
    return pl.pallas_call(
        _de_write_body,
        grid=(BATCH - SC_BATCHES,),
        in_specs=[
            pl.BlockSpec(memory_space=pl.ANY),
            pl.BlockSpec((CORPUS, H), lambda b: (0, 0)),
        ],
        out_specs=pl.BlockSpec((1, CORPUS, H), lambda b: (b + SC_BATCHES, 0, 0)),
        out_shape=jax.ShapeDtypeStruct((BATCH, ROWS, H), jnp.float32),
        input_output_aliases={0: 0},
    )(out1, de)


def kernel(input_ids, dataset_embeddings, W1, b1, W2, b2):
    del input_ids
    de = dataset_embeddings.astype(jnp.float32)
    sp = _soft_prompt(W1, b1, W2, b2)
    out = _sc_broadcast_de(de)
    out = _tc_de_write(out, de)
    out = _sp_write(out, sp)
    return out
